# baseline (speedup 1.0000x reference)
import jax
import jax.numpy as jnp
from jax.experimental import pallas as pl
from jax.experimental.pallas import tpu as pltpu

BN_EPS = 1e-5
N = 128
VMEM_LIMIT = 110 * 1024 * 1024


def _elu(x):
    one = jnp.asarray(1.0, x.dtype)
    return jnp.maximum(x, jnp.exp(jnp.minimum(x, 0)) - one)


def _fused_kernel(x_ref, sr_ref, offd_ref,
                  we1_ref, be1_ref, we2_ref, be2_ref,
                  w1sr1_ref, b11_ref, w21_ref, b21_ref,
                  wn1_ref, bn1_ref, wn2_ref, bn2_ref,
                  w1sr2_ref, w1k2_ref, b12_ref, w22_ref, b22_ref,
                  sc2_ref, sh2_ref,
                  o_ref, sq_ref):
    f32 = jnp.float32
    bf16 = jnp.bfloat16

    xin = x_ref[0].astype(bf16)
    h = _elu(jnp.dot(xin, we1_ref[...], preferred_element_type=f32)
             + be1_ref[...])
    y = _elu(jnp.dot(h.astype(bf16), we2_ref[...], preferred_element_type=f32)
             + be2_ref[...])
    x = y.astype(bf16)

    xsr = jnp.dot(x, w1sr1_ref[...], preferred_element_type=f32)
    H = xsr.shape[1] // 2
    xs = xsr[:, :H]
    xrb = xsr[:, H:] + b11_ref[...]
    xstack = jnp.concatenate([xs, xrb], axis=0).astype(bf16)
    pre1 = jnp.dot(sr_ref[...], xstack, preferred_element_type=f32)
    h1 = _elu(pre1.astype(bf16))

    m1 = jnp.dot(h1, w21_ref[...], preferred_element_type=f32)
    msg = _elu((m1 + b21_ref[...]).astype(bf16))

    aggraw = jnp.dot(offd_ref[...], msg, preferred_element_type=f32)

    hn = _elu(jnp.dot(aggraw.astype(bf16), wn1_ref[...],
                      preferred_element_type=f32) + bn1_ref[...])
    yn = _elu(jnp.dot(hn.astype(bf16), wn2_ref[...],
                      preferred_element_type=f32) + bn2_ref[...])
    xn = yn.astype(bf16)

    xnsr = jnp.dot(xn, w1sr2_ref[...], preferred_element_type=f32)
    H2 = xnsr.shape[1] // 2
    xnstack = jnp.concatenate([xnsr[:, :H2], xnsr[:, H2:] + b12_ref[...]],
                              axis=0).astype(bf16)
    pre2 = (jnp.dot(sr_ref[...], xnstack, preferred_element_type=f32)
            + jnp.dot(msg, w1k2_ref[...], preferred_element_type=f32))
    h2 = _elu(pre2.astype(bf16))

    y2 = jnp.dot(h2, w22_ref[...], preferred_element_type=f32) + b22_ref[...]
    sq_ref[...] = (_elu(y2) * sc2_ref[...] + sh2_ref[...]).reshape(
        N, N, y2.shape[1])

    head = sq_ref[:, : N - 1, :]
    tail = sq_ref[:, 1:, :]
    ik = jax.lax.broadcasted_iota(jnp.int32, (N, N - 1, 1), 0)
    kk = jax.lax.broadcasted_iota(jnp.int32, (N, N - 1, 1), 1)
    o_ref[0] = jnp.where(kk < ik, head, tail)


def kernel(emb_w1, emb_b1, emb_w2, emb_b2, emb_gamma, emb_beta,
           e1_w1, e1_b1, e1_w2, e1_b2, e1_gamma, e1_beta,
           n1_w1, n1_b1, n1_w2, n1_b2, n1_gamma, n1_beta,
           e2_w1, e2_b1, e2_w2, e2_b2, e2_gamma, e2_beta,
           inputs, rel_rec, rel_send):
    f32 = jnp.float32
    bf16 = jnp.bfloat16
    B, n_atoms, n_in = inputs.shape
    assert n_atoms == N
    D = emb_w2.shape[1]
    Dn = n1_w2.shape[1]
    Dout = e2_w2.shape[1]
    E = N * (N - 1)

    sq = jnp.sqrt(jnp.asarray(1.0 + BN_EPS, f32))
    sce, she = emb_gamma / sq, emb_beta
    sc1, sh1 = e1_gamma / sq, e1_beta
    scn, shn = n1_gamma / sq, n1_beta
    sc2, sh2 = e2_gamma / sq, e2_beta

    eye = jnp.eye(N, dtype=bf16)
    send_oh = jnp.tile(eye, (N, 1))
    recv_oh = jnp.repeat(eye, N, axis=0)
    sr = jnp.concatenate([send_oh, recv_oh], axis=1)
    offd = (jnp.repeat(eye, N, axis=1)
            * (1.0 - eye.reshape(1, N * N))).astype(bf16)

    w1sr1 = jnp.concatenate([e1_w1[:D], e1_w1[D:]], axis=1)
    w1sr1_eff = sce[:, None] * w1sr1
    b11_eff = e1_b1 + (she @ w1sr1)[:D] + (she @ w1sr1)[D:]
    wn1_eff = (sc1[:, None] * n1_w1) / float(N)
    bn1_eff = n1_b1 + (N - 1) / float(N) * (sh1 @ n1_w1)
    w1sr2 = jnp.concatenate([e2_w1[:Dn], e2_w1[Dn:2 * Dn]], axis=1)
    w1sr2_eff = scn[:, None] * w1sr2
    w1k_eff = sc1[:, None] * e2_w1[2 * Dn:]
    b12_eff = (e2_b1 + sh1 @ e2_w1[2 * Dn:]
               + (shn @ w1sr2)[:Dn] + (shn @ w1sr2)[Dn:])

    args = (
        inputs, sr, offd,
        emb_w1.astype(bf16), emb_b1.reshape(1, -1),
        emb_w2.astype(bf16), emb_b2.reshape(1, -1),
        w1sr1_eff.astype(bf16), b11_eff.reshape(1, -1),
        e1_w2.astype(bf16), e1_b2.reshape(1, -1),
        wn1_eff.astype(bf16), bn1_eff.reshape(1, -1),
        n1_w2.astype(bf16), n1_b2.reshape(1, -1),
        w1sr2_eff.astype(bf16), w1k_eff.astype(bf16), b12_eff.reshape(1, -1),
        e2_w2.astype(bf16), e2_b2.reshape(1, -1),
        sc2.reshape(1, -1), sh2.reshape(1, -1),
    )

    const2 = lambda b: (0, 0)
    in_specs = [pl.BlockSpec((1, N, n_in), lambda b: (b, 0, 0))]
    in_specs += [pl.BlockSpec(a.shape, const2) for a in args[1:]]

    out = pl.pallas_call(
        _fused_kernel,
        out_shape=jax.ShapeDtypeStruct((B, N, N - 1, Dout), f32),
        grid=(B,),
        in_specs=in_specs,
        out_specs=pl.BlockSpec((1, N, N - 1, Dout), lambda b: (b, 0, 0, 0)),
        scratch_shapes=[pltpu.VMEM((N, N, Dout), f32)],
        compiler_params=pltpu.CompilerParams(
            dimension_semantics=("parallel",),
            vmem_limit_bytes=VMEM_LIMIT),
    )(*args)
    return out.reshape(B, E, Dout)


# speedup vs baseline: 1.8455x; 1.8455x over previous
import jax
import jax.numpy as jnp
from jax.experimental import pallas as pl
from jax.experimental.pallas import tpu as pltpu

BN_EPS = 1e-5
N = 128
VMEM_LIMIT = 110 * 1024 * 1024


def _elu(x):
    one = jnp.asarray(1.0, x.dtype)
    return jnp.maximum(x, jnp.exp(jnp.minimum(x, 0)) - one)


def _fused_kernel(x_ref, sr_ref, offd_ref,
                  we1_ref, be1_ref, we2_ref, be2_ref,
                  w1sr1_ref, b11_ref, w21_ref, b21_ref,
                  wn1_ref, bn1_ref, wn2_ref, bn2_ref,
                  w1sr2_ref, w1k2_ref, b12_ref, w22_ref, b22_ref,
                  sc2_ref, sh2_ref,
                  o_ref, sq_ref):
    f32 = jnp.float32
    bf16 = jnp.bfloat16

    xin = x_ref[0].astype(bf16)
    h = _elu(jnp.dot(xin, we1_ref[...], preferred_element_type=f32)
             + be1_ref[...])
    y = _elu(jnp.dot(h.astype(bf16), we2_ref[...], preferred_element_type=f32)
             + be2_ref[...])
    x = y.astype(bf16)

    xsr = jnp.dot(x, w1sr1_ref[...], preferred_element_type=f32)
    H = xsr.shape[1] // 2
    xs = xsr[:, :H]
    xrb = xsr[:, H:] + b11_ref[...]
    xstack = jnp.concatenate([xs, xrb], axis=0).astype(bf16)
    pre1 = jnp.dot(sr_ref[...], xstack, preferred_element_type=f32)
    h1 = _elu(pre1.astype(bf16))

    m1 = jnp.dot(h1, w21_ref[...], preferred_element_type=f32)
    msg = _elu(m1.astype(bf16) + b21_ref[...])

    aggraw = jnp.dot(offd_ref[...], msg, preferred_element_type=f32)

    hn = _elu(jnp.dot(aggraw.astype(bf16), wn1_ref[...],
                      preferred_element_type=f32) + bn1_ref[...])
    yn = _elu(jnp.dot(hn.astype(bf16), wn2_ref[...],
                      preferred_element_type=f32) + bn2_ref[...])
    xn = yn.astype(bf16)

    xnsr = jnp.dot(xn, w1sr2_ref[...], preferred_element_type=f32)
    H2 = xnsr.shape[1] // 2
    xnstack = jnp.concatenate([xnsr[:, :H2], xnsr[:, H2:] + b12_ref[...]],
                              axis=0).astype(bf16)
    pre2 = (jnp.dot(sr_ref[...], xnstack, preferred_element_type=f32)
            + jnp.dot(msg, w1k2_ref[...], preferred_element_type=f32))
    h2 = _elu(pre2.astype(bf16))

    y2 = jnp.dot(h2, w22_ref[...], preferred_element_type=f32) + b22_ref[...]
    sq_ref[...] = _elu(y2) * sc2_ref[...] + sh2_ref[...]

    for g in range(N - 1):
        o_ref[0, pl.ds(g * N, N), :] = sq_ref[pl.ds(g * (N + 1) + 1, N), :]


def kernel(emb_w1, emb_b1, emb_w2, emb_b2, emb_gamma, emb_beta,
           e1_w1, e1_b1, e1_w2, e1_b2, e1_gamma, e1_beta,
           n1_w1, n1_b1, n1_w2, n1_b2, n1_gamma, n1_beta,
           e2_w1, e2_b1, e2_w2, e2_b2, e2_gamma, e2_beta,
           inputs, rel_rec, rel_send):
    f32 = jnp.float32
    bf16 = jnp.bfloat16
    B, n_atoms, n_in = inputs.shape
    assert n_atoms == N
    D = emb_w2.shape[1]
    Dn = n1_w2.shape[1]
    Dout = e2_w2.shape[1]
    E = N * (N - 1)

    sq = jnp.sqrt(jnp.asarray(1.0 + BN_EPS, f32))
    sce, she = emb_gamma / sq, emb_beta
    sc1, sh1 = e1_gamma / sq, e1_beta
    scn, shn = n1_gamma / sq, n1_beta
    sc2, sh2 = e2_gamma / sq, e2_beta

    eye = jnp.eye(N, dtype=bf16)
    send_oh = jnp.tile(eye, (N, 1))
    recv_oh = jnp.repeat(eye, N, axis=0)
    sr = jnp.concatenate([send_oh, recv_oh], axis=1)
    offd = (jnp.repeat(eye, N, axis=1)
            * (1.0 - eye.reshape(1, N * N))).astype(bf16)

    w1sr1 = jnp.concatenate([e1_w1[:D], e1_w1[D:]], axis=1)
    w1sr1_eff = sce[:, None] * w1sr1
    b11_eff = e1_b1 + (she @ w1sr1)[:D] + (she @ w1sr1)[D:]
    wn1_eff = (sc1[:, None] * n1_w1) / float(N)
    bn1_eff = n1_b1 + (N - 1) / float(N) * (sh1 @ n1_w1)
    w1sr2 = jnp.concatenate([e2_w1[:Dn], e2_w1[Dn:2 * Dn]], axis=1)
    w1sr2_eff = scn[:, None] * w1sr2
    w1k_eff = sc1[:, None] * e2_w1[2 * Dn:]
    b12_eff = (e2_b1 + sh1 @ e2_w1[2 * Dn:]
               + (shn @ w1sr2)[:Dn] + (shn @ w1sr2)[Dn:])

    args = (
        inputs, sr, offd,
        emb_w1.astype(bf16), emb_b1.reshape(1, -1),
        emb_w2.astype(bf16), emb_b2.reshape(1, -1),
        w1sr1_eff.astype(bf16), b11_eff.reshape(1, -1),
        e1_w2.astype(bf16), e1_b2.reshape(1, -1).astype(bf16),
        wn1_eff.astype(bf16), bn1_eff.reshape(1, -1),
        n1_w2.astype(bf16), n1_b2.reshape(1, -1),
        w1sr2_eff.astype(bf16), w1k_eff.astype(bf16), b12_eff.reshape(1, -1),
        e2_w2.astype(bf16), e2_b2.reshape(1, -1),
        sc2.reshape(1, -1), sh2.reshape(1, -1),
    )

    const2 = lambda b: (0, 0)
    in_specs = [pl.BlockSpec((1, N, n_in), lambda b: (b, 0, 0))]
    in_specs += [pl.BlockSpec(a.shape, const2) for a in args[1:]]

    return pl.pallas_call(
        _fused_kernel,
        out_shape=jax.ShapeDtypeStruct((B, E, Dout), f32),
        grid=(B,),
        in_specs=in_specs,
        out_specs=pl.BlockSpec((1, E, Dout), lambda b: (b, 0, 0)),
        scratch_shapes=[pltpu.VMEM((N * N, Dout), f32)],
        compiler_params=pltpu.CompilerParams(
            dimension_semantics=("parallel",),
            vmem_limit_bytes=VMEM_LIMIT),
    )(*args)


# speedup vs baseline: 1.9280x; 1.0447x over previous
import jax
import jax.numpy as jnp
from jax.experimental import pallas as pl
from jax.experimental.pallas import tpu as pltpu

BN_EPS = 1e-5
VMEM_LIMIT = 110 * 1024 * 1024


def _elu(x):
    one = jnp.asarray(1.0, x.dtype)
    return jnp.maximum(x, jnp.exp(jnp.minimum(x, 0)) - one)


def _fused_kernel(x_ref, src_ref, rt_ref,
                  we1_ref, be1_ref, we2_ref, be2_ref,
                  w1sr1_ref, b11_ref, w21_ref, b21_ref,
                  wn1_ref, bn1_ref, wn2_ref, bn2_ref,
                  w1sr2_ref, w1k2_ref, b12_ref, w22_ref, b22_ref,
                  sc2_ref, sh2_ref,
                  o_ref):
    f32 = jnp.float32
    bf16 = jnp.bfloat16
    N = x_ref.shape[1]

    xin = x_ref[0].astype(bf16)
    h = _elu(jnp.dot(xin, we1_ref[...], preferred_element_type=f32)
             + be1_ref[...])
    y = _elu(jnp.dot(h.astype(bf16), we2_ref[...], preferred_element_type=f32)
             + be2_ref[...])
    x = y.astype(bf16)

    xsr = jnp.dot(x, w1sr1_ref[...], preferred_element_type=f32)
    H = xsr.shape[1] // 2
    xs = xsr[:, :H]
    xrb = xsr[:, H:] + b11_ref[...]
    xstack = jnp.concatenate([xs, xrb], axis=0).astype(bf16)
    pre1 = jnp.dot(src_ref[...], xstack, preferred_element_type=f32)
    h1 = _elu(pre1.astype(bf16))

    m1 = jnp.dot(h1, w21_ref[...], preferred_element_type=f32)
    msg = _elu(m1.astype(bf16) + b21_ref[...])

    aggraw = jnp.dot(rt_ref[...], msg, preferred_element_type=f32)

    hn = _elu(jnp.dot(aggraw.astype(bf16), wn1_ref[...],
                      preferred_element_type=f32) + bn1_ref[...])
    yn = _elu(jnp.dot(hn.astype(bf16), wn2_ref[...],
                      preferred_element_type=f32) + bn2_ref[...])
    xn = yn.astype(bf16)

    xnsr = jnp.dot(xn, w1sr2_ref[...], preferred_element_type=f32)
    H2 = xnsr.shape[1] // 2
    xnstack = jnp.concatenate([xnsr[:, :H2], xnsr[:, H2:] + b12_ref[...]],
                              axis=0).astype(bf16)
    pre2 = (jnp.dot(src_ref[...], xnstack, preferred_element_type=f32)
            + jnp.dot(msg, w1k2_ref[...], preferred_element_type=f32))
    h2 = _elu(pre2.astype(bf16))

    y2 = jnp.dot(h2, w22_ref[...], preferred_element_type=f32) + b22_ref[...]
    o_ref[0] = _elu(y2) * sc2_ref[...] + sh2_ref[...]


def kernel(emb_w1, emb_b1, emb_w2, emb_b2, emb_gamma, emb_beta,
           e1_w1, e1_b1, e1_w2, e1_b2, e1_gamma, e1_beta,
           n1_w1, n1_b1, n1_w2, n1_b2, n1_gamma, n1_beta,
           e2_w1, e2_b1, e2_w2, e2_b2, e2_gamma, e2_beta,
           inputs, rel_rec, rel_send):
    f32 = jnp.float32
    bf16 = jnp.bfloat16
    B, N, n_in = inputs.shape
    E = rel_rec.shape[0]
    D = emb_w2.shape[1]
    Dn = n1_w2.shape[1]
    Dout = e2_w2.shape[1]

    sq = jnp.sqrt(jnp.asarray(1.0 + BN_EPS, f32))
    sce, she = emb_gamma / sq, emb_beta
    sc1, sh1 = e1_gamma / sq, e1_beta
    scn, shn = n1_gamma / sq, n1_beta
    sc2, sh2 = e2_gamma / sq, e2_beta

    src_cat = jnp.concatenate([rel_send, rel_rec], axis=1).astype(bf16)
    rt = rel_rec.T.astype(bf16)

    w1sr1 = jnp.concatenate([e1_w1[:D], e1_w1[D:]], axis=1)
    w1sr1_eff = sce[:, None] * w1sr1
    b11_eff = e1_b1 + (she @ w1sr1)[:D] + (she @ w1sr1)[D:]
    wn1_eff = (sc1[:, None] * n1_w1) / float(N)
    bn1_eff = n1_b1 + (N - 1) / float(N) * (sh1 @ n1_w1)
    w1sr2 = jnp.concatenate([e2_w1[:Dn], e2_w1[Dn:2 * Dn]], axis=1)
    w1sr2_eff = scn[:, None] * w1sr2
    w1k_eff = sc1[:, None] * e2_w1[2 * Dn:]
    b12_eff = (e2_b1 + sh1 @ e2_w1[2 * Dn:]
               + (shn @ w1sr2)[:Dn] + (shn @ w1sr2)[Dn:])

    args = (
        inputs, src_cat, rt,
        emb_w1.astype(bf16), emb_b1.reshape(1, -1),
        emb_w2.astype(bf16), emb_b2.reshape(1, -1),
        w1sr1_eff.astype(bf16), b11_eff.reshape(1, -1),
        e1_w2.astype(bf16), e1_b2.reshape(1, -1).astype(bf16),
        wn1_eff.astype(bf16), bn1_eff.reshape(1, -1),
        n1_w2.astype(bf16), n1_b2.reshape(1, -1),
        w1sr2_eff.astype(bf16), w1k_eff.astype(bf16), b12_eff.reshape(1, -1),
        e2_w2.astype(bf16), e2_b2.reshape(1, -1),
        sc2.reshape(1, -1), sh2.reshape(1, -1),
    )

    const2 = lambda b: (0, 0)
    in_specs = [pl.BlockSpec((1, N, n_in), lambda b: (b, 0, 0))]
    in_specs += [pl.BlockSpec(a.shape, const2) for a in args[1:]]

    return pl.pallas_call(
        _fused_kernel,
        out_shape=jax.ShapeDtypeStruct((B, E, Dout), f32),
        grid=(B,),
        in_specs=in_specs,
        out_specs=pl.BlockSpec((1, E, Dout), lambda b: (b, 0, 0)),
        compiler_params=pltpu.CompilerParams(
            dimension_semantics=("parallel",),
            vmem_limit_bytes=VMEM_LIMIT),
    )(*args)


# speedup vs baseline: 2.0711x; 1.0742x over previous
import jax
import jax.numpy as jnp
from jax.experimental import pallas as pl
from jax.experimental.pallas import tpu as pltpu

BN_EPS = 1e-5
VMEM_LIMIT = 110 * 1024 * 1024


def _elu(x):
    one = jnp.asarray(1.0, x.dtype)
    return jnp.maximum(x, jnp.exp(jnp.minimum(x, 0)) - one)


def _fused_kernel(x_ref, src_ref, rt_ref,
                  we1_ref, be1_ref, we2_ref, be2_ref,
                  w1sr1_ref, b11_ref, w21_ref, b21_ref,
                  wn1_ref, bn1_ref, wn2_ref, bn2_ref,
                  w1sr2_ref, w1k2_ref, b12_ref, w22_ref, b22_ref,
                  sc2_ref, sh2_ref,
                  o_ref, res_ref):
    f32 = jnp.float32
    bf16 = jnp.bfloat16
    N = x_ref.shape[1]
    t = pl.program_id(1)

    @pl.when(t == 0)
    def _compute():
        xin = x_ref[...].reshape(2 * N, x_ref.shape[2]).astype(bf16)
        h = _elu(jnp.dot(xin, we1_ref[...], preferred_element_type=f32)
                 + be1_ref[...])
        y = _elu(jnp.dot(h.astype(bf16), we2_ref[...],
                         preferred_element_type=f32) + be2_ref[...])
        x = y.astype(bf16)

        xsr = jnp.dot(x, w1sr1_ref[...], preferred_element_type=f32)
        H = xsr.shape[1] // 2
        ca = jnp.concatenate([xsr[:N, :H], xsr[:N, H:] + b11_ref[...]],
                             axis=0)
        cb = jnp.concatenate([xsr[N:, :H], xsr[N:, H:] + b11_ref[...]],
                             axis=0)
        xstack = jnp.concatenate([ca, cb], axis=1).astype(bf16)
        pre1 = jnp.dot(src_ref[...], xstack, preferred_element_type=f32)
        h1 = _elu(pre1.astype(bf16))

        m1 = jnp.dot(h1, w21_ref[...], preferred_element_type=f32)
        msg = _elu(m1.astype(bf16) + b21_ref[...])

        aggraw = jnp.dot(rt_ref[...], msg, preferred_element_type=f32)

        hn = _elu(jnp.dot(aggraw.astype(bf16), wn1_ref[...],
                          preferred_element_type=f32) + bn1_ref[...])
        yn = _elu(jnp.dot(hn.astype(bf16), wn2_ref[...],
                          preferred_element_type=f32) + bn2_ref[...])
        xn = yn.astype(bf16)

        xnsr = jnp.dot(xn, w1sr2_ref[...], preferred_element_type=f32)
        H4 = xnsr.shape[1] // 4
        da = jnp.concatenate([xnsr[:, :H4], xnsr[:, H4:2 * H4]
                              + b12_ref[...]], axis=0)
        db = jnp.concatenate([xnsr[:, 2 * H4:3 * H4], xnsr[:, 3 * H4:]
                              + b12_ref[...]], axis=0)
        xnstack = jnp.concatenate([da, db], axis=1).astype(bf16)
        pre2 = (jnp.dot(src_ref[...], xnstack, preferred_element_type=f32)
                + jnp.dot(msg, w1k2_ref[...], preferred_element_type=f32))
        h2 = _elu(pre2.astype(bf16))

        y2 = (jnp.dot(h2, w22_ref[...], preferred_element_type=f32)
              + b22_ref[...])
        res = _elu(y2) * sc2_ref[...] + sh2_ref[...]
        Dout = res.shape[1] // 2
        o_ref[0] = res[:, :Dout]
        res_ref[...] = res[:, Dout:]

    @pl.when(t == 1)
    def _flush():
        o_ref[0] = res_ref[...]


def kernel(emb_w1, emb_b1, emb_w2, emb_b2, emb_gamma, emb_beta,
           e1_w1, e1_b1, e1_w2, e1_b2, e1_gamma, e1_beta,
           n1_w1, n1_b1, n1_w2, n1_b2, n1_gamma, n1_beta,
           e2_w1, e2_b1, e2_w2, e2_b2, e2_gamma, e2_beta,
           inputs, rel_rec, rel_send):
    f32 = jnp.float32
    bf16 = jnp.bfloat16
    B, N, n_in = inputs.shape
    E = rel_rec.shape[0]
    D = emb_w2.shape[1]
    Dn = n1_w2.shape[1]
    Dout = e2_w2.shape[1]

    sq = jnp.sqrt(jnp.asarray(1.0 + BN_EPS, f32))
    sce, she = emb_gamma / sq, emb_beta
    sc1, sh1 = e1_gamma / sq, e1_beta
    scn, shn = n1_gamma / sq, n1_beta
    sc2, sh2 = e2_gamma / sq, e2_beta

    src_cat = jnp.concatenate([rel_send, rel_rec], axis=1).astype(bf16)
    rt = rel_rec.T.astype(bf16)

    w1sr1 = jnp.concatenate([e1_w1[:D], e1_w1[D:]], axis=1)
    w1sr1_eff = sce[:, None] * w1sr1
    b11_eff = e1_b1 + (she @ w1sr1)[:D] + (she @ w1sr1)[D:]
    wn1_eff = (sc1[:, None] * n1_w1) / float(N)
    bn1_eff = n1_b1 + (N - 1) / float(N) * (sh1 @ n1_w1)
    w1sr2 = jnp.concatenate([e2_w1[:Dn], e2_w1[Dn:2 * Dn]], axis=1)
    w1sr2_eff = scn[:, None] * w1sr2
    w1k_eff = sc1[:, None] * e2_w1[2 * Dn:]
    b12_eff = (e2_b1 + sh1 @ e2_w1[2 * Dn:]
               + (shn @ w1sr2)[:Dn] + (shn @ w1sr2)[Dn:])

    def bdiag(w):
        z = jnp.zeros_like(w)
        return jnp.block([[w, z], [z, w]])

    pair = lambda v: jnp.tile(v.reshape(1, -1), (1, 2))

    args = (
        inputs, src_cat, rt,
        emb_w1.astype(bf16), emb_b1.reshape(1, -1),
        emb_w2.astype(bf16), emb_b2.reshape(1, -1),
        w1sr1_eff.astype(bf16), b11_eff.reshape(1, -1),
        bdiag(e1_w2).astype(bf16), pair(e1_b2).astype(bf16),
        bdiag(wn1_eff).astype(bf16), pair(bn1_eff),
        bdiag(n1_w2).astype(bf16), pair(n1_b2),
        bdiag(w1sr2_eff).astype(bf16), bdiag(w1k_eff).astype(bf16),
        b12_eff.reshape(1, -1),
        bdiag(e2_w2).astype(bf16), pair(e2_b2),
        pair(sc2), pair(sh2),
    )

    const2 = lambda p, t: (0, 0)
    in_specs = [pl.BlockSpec((2, N, n_in), lambda p, t: (p, 0, 0))]
    in_specs += [pl.BlockSpec(a.shape, const2) for a in args[1:]]

    return pl.pallas_call(
        _fused_kernel,
        out_shape=jax.ShapeDtypeStruct((B, E, Dout), f32),
        grid=(B // 2, 2),
        in_specs=in_specs,
        out_specs=pl.BlockSpec((1, E, Dout), lambda p, t: (2 * p + t, 0, 0)),
        scratch_shapes=[pltpu.VMEM((E, Dout), f32)],
        compiler_params=pltpu.CompilerParams(
            dimension_semantics=("parallel", "arbitrary"),
            vmem_limit_bytes=VMEM_LIMIT),
    )(*args)


# speedup vs baseline: 2.2345x; 1.0789x over previous
import jax
import jax.numpy as jnp
from jax.experimental import pallas as pl
from jax.experimental.pallas import tpu as pltpu

BN_EPS = 1e-5
VMEM_LIMIT = 110 * 1024 * 1024


def _elu(x):
    one = jnp.asarray(1.0, x.dtype)
    return jnp.maximum(x, jnp.exp(jnp.minimum(x, 0)) - one)


def _fused_kernel(x_ref, src_ref, rt_ref,
                  we1_ref, be1_ref, we2_ref, be2_ref,
                  w1sr1_ref, b11_ref, w21_ref, b21_ref,
                  wn1_ref, bn1_ref, wn2_ref, bn2_ref,
                  w1sr2_ref, w1k2_ref, b12_ref, w22_ref, b22_ref,
                  sc2_ref, sh2_ref,
                  o_ref, res_ref):
    f32 = jnp.float32
    bf16 = jnp.bfloat16
    N = x_ref.shape[1]
    t = pl.program_id(1)

    @pl.when(t == 0)
    def _compute():
        xin = x_ref[...].reshape(2 * N, x_ref.shape[2]).astype(bf16)
        h = _elu(jnp.dot(xin, we1_ref[...], preferred_element_type=f32)
                 + be1_ref[...])
        y = _elu(jnp.dot(h.astype(bf16), we2_ref[...],
                         preferred_element_type=f32) + be2_ref[...])
        x = y.astype(bf16)

        xsr = jnp.dot(x, w1sr1_ref[...], preferred_element_type=f32)
        H = xsr.shape[1] // 2
        ca = jnp.concatenate([xsr[:N, :H], xsr[:N, H:] + b11_ref[...]],
                             axis=0)
        cb = jnp.concatenate([xsr[N:, :H], xsr[N:, H:] + b11_ref[...]],
                             axis=0)
        xstack = jnp.concatenate([ca, cb], axis=1).astype(bf16)
        pre1 = jnp.dot(src_ref[...], xstack, preferred_element_type=f32)
        h1 = _elu(pre1.astype(bf16))

        m1 = jnp.dot(h1, w21_ref[...], preferred_element_type=f32)
        msg = _elu(m1.astype(bf16) + b21_ref[...])

        aggraw = jnp.dot(rt_ref[...], msg, preferred_element_type=f32)

        hn = _elu(jnp.dot(aggraw.astype(bf16), wn1_ref[...],
                          preferred_element_type=f32) + bn1_ref[...])
        yn = _elu(jnp.dot(hn.astype(bf16), wn2_ref[...],
                          preferred_element_type=f32) + bn2_ref[...])
        xn = yn.astype(bf16)

        xnsr = jnp.dot(xn, w1sr2_ref[...], preferred_element_type=f32)
        H4 = xnsr.shape[1] // 4
        da = jnp.concatenate([xnsr[:, :H4], xnsr[:, H4:2 * H4]
                              + b12_ref[...]], axis=0)
        db = jnp.concatenate([xnsr[:, 2 * H4:3 * H4], xnsr[:, 3 * H4:]
                              + b12_ref[...]], axis=0)
        xnstack = jnp.concatenate([da, db], axis=1).astype(bf16)
        pre2 = (jnp.dot(src_ref[...], xnstack, preferred_element_type=f32)
                + jnp.dot(msg, w1k2_ref[...], preferred_element_type=f32))
        h2 = _elu(pre2.astype(bf16))

        y2 = jnp.dot(h2, w22_ref[...], preferred_element_type=f32)
        res = (_elu(y2.astype(bf16) + b22_ref[...]) * sc2_ref[...]
               + sh2_ref[...])
        Dout = res.shape[1] // 2
        o_ref[0] = res[:, :Dout].astype(f32)
        res_ref[...] = res[:, Dout:]

    @pl.when(t == 1)
    def _flush():
        o_ref[0] = res_ref[...].astype(jnp.float32)


def kernel(emb_w1, emb_b1, emb_w2, emb_b2, emb_gamma, emb_beta,
           e1_w1, e1_b1, e1_w2, e1_b2, e1_gamma, e1_beta,
           n1_w1, n1_b1, n1_w2, n1_b2, n1_gamma, n1_beta,
           e2_w1, e2_b1, e2_w2, e2_b2, e2_gamma, e2_beta,
           inputs, rel_rec, rel_send):
    f32 = jnp.float32
    bf16 = jnp.bfloat16
    B, N, n_in = inputs.shape
    E = rel_rec.shape[0]
    D = emb_w2.shape[1]
    Dn = n1_w2.shape[1]
    Dout = e2_w2.shape[1]

    sq = jnp.sqrt(jnp.asarray(1.0 + BN_EPS, f32))
    sce, she = emb_gamma / sq, emb_beta
    sc1, sh1 = e1_gamma / sq, e1_beta
    scn, shn = n1_gamma / sq, n1_beta
    sc2, sh2 = e2_gamma / sq, e2_beta

    src_cat = jnp.concatenate([rel_send, rel_rec], axis=1).astype(bf16)
    rt = rel_rec.T.astype(bf16)

    w1sr1 = jnp.concatenate([e1_w1[:D], e1_w1[D:]], axis=1)
    w1sr1_eff = sce[:, None] * w1sr1
    b11_eff = e1_b1 + (she @ w1sr1)[:D] + (she @ w1sr1)[D:]
    wn1_eff = (sc1[:, None] * n1_w1) / float(N)
    bn1_eff = n1_b1 + (N - 1) / float(N) * (sh1 @ n1_w1)
    w1sr2 = jnp.concatenate([e2_w1[:Dn], e2_w1[Dn:2 * Dn]], axis=1)
    w1sr2_eff = scn[:, None] * w1sr2
    w1k_eff = sc1[:, None] * e2_w1[2 * Dn:]
    b12_eff = (e2_b1 + sh1 @ e2_w1[2 * Dn:]
               + (shn @ w1sr2)[:Dn] + (shn @ w1sr2)[Dn:])

    def bdiag(w):
        z = jnp.zeros_like(w)
        return jnp.block([[w, z], [z, w]])

    pair = lambda v: jnp.tile(v.reshape(1, -1), (1, 2))

    args = (
        inputs, src_cat, rt,
        emb_w1.astype(bf16), emb_b1.reshape(1, -1),
        emb_w2.astype(bf16), emb_b2.reshape(1, -1),
        w1sr1_eff.astype(bf16), b11_eff.reshape(1, -1),
        bdiag(e1_w2).astype(bf16), pair(e1_b2).astype(bf16),
        bdiag(wn1_eff).astype(bf16), pair(bn1_eff),
        bdiag(n1_w2).astype(bf16), pair(n1_b2),
        bdiag(w1sr2_eff).astype(bf16), bdiag(w1k_eff).astype(bf16),
        b12_eff.reshape(1, -1),
        bdiag(e2_w2).astype(bf16), pair(e2_b2).astype(bf16),
        pair(sc2).astype(bf16), pair(sh2).astype(bf16),
    )

    const2 = lambda p, t: (0, 0)
    in_specs = [pl.BlockSpec((2, N, n_in), lambda p, t: (p, 0, 0))]
    in_specs += [pl.BlockSpec(a.shape, const2) for a in args[1:]]

    return pl.pallas_call(
        _fused_kernel,
        out_shape=jax.ShapeDtypeStruct((B, E, Dout), f32),
        grid=(B // 2, 2),
        in_specs=in_specs,
        out_specs=pl.BlockSpec((1, E, Dout), lambda p, t: (2 * p + t, 0, 0)),
        scratch_shapes=[pltpu.VMEM((E, Dout), bf16)],
        compiler_params=pltpu.CompilerParams(
            dimension_semantics=("parallel", "arbitrary"),
            vmem_limit_bytes=VMEM_LIMIT),
    )(*args)


# speedup vs baseline: 2.2347x; 1.0001x over previous
import jax
import jax.numpy as jnp
from jax.experimental import pallas as pl
from jax.experimental.pallas import tpu as pltpu

BN_EPS = 1e-5
VMEM_LIMIT = 110 * 1024 * 1024


def _elu(x):
    one = jnp.asarray(1.0, x.dtype)
    return jnp.maximum(x, jnp.exp(jnp.minimum(x, 0)) - one)


def _fused_kernel(x_ref, src_ref, rt_ref,
                  we1_ref, be1_ref, we2_ref, be2_ref,
                  w1sr1_ref, b11_ref, w21_ref, b21_ref,
                  wn1_ref, bn1_ref, wn2_ref, bn2_ref,
                  w1sr2_ref, w1k2_ref, b12_ref, w22_ref, b22_ref,
                  sc2_ref, sh2_ref,
                  o_ref, res_ref):
    f32 = jnp.float32
    bf16 = jnp.bfloat16
    N = x_ref.shape[1]
    t = pl.program_id(1)

    @pl.when(t == 0)
    def _compute():
        xin = x_ref[...].reshape(2 * N, x_ref.shape[2]).astype(bf16)
        h = _elu(jnp.dot(xin, we1_ref[...], preferred_element_type=f32)
                 + be1_ref[...])
        y = _elu(jnp.dot(h.astype(bf16), we2_ref[...],
                         preferred_element_type=f32) + be2_ref[...])
        x = y.astype(bf16)

        xsr = jnp.dot(x, w1sr1_ref[...], preferred_element_type=f32)
        H = xsr.shape[1] // 2
        ca = jnp.concatenate([xsr[:N, :H], xsr[:N, H:] + b11_ref[...]],
                             axis=0)
        cb = jnp.concatenate([xsr[N:, :H], xsr[N:, H:] + b11_ref[...]],
                             axis=0)
        xstack = jnp.concatenate([ca, cb], axis=1).astype(bf16)
        pre1 = jnp.dot(src_ref[...], xstack, preferred_element_type=f32)
        h1 = _elu(pre1.astype(bf16))

        m1 = jnp.dot(h1, w21_ref[...], preferred_element_type=f32)
        msg = _elu(m1.astype(bf16) + b21_ref[...])

        aggraw = jnp.dot(rt_ref[...], msg, preferred_element_type=f32)

        hn = _elu(jnp.dot(aggraw.astype(bf16), wn1_ref[...],
                          preferred_element_type=f32) + bn1_ref[...])
        yn = _elu(jnp.dot(hn.astype(bf16), wn2_ref[...],
                          preferred_element_type=f32) + bn2_ref[...])
        xn = yn.astype(bf16)

        xnsr = jnp.dot(xn, w1sr2_ref[...], preferred_element_type=f32)
        H4 = xnsr.shape[1] // 4
        da = jnp.concatenate([xnsr[:, :H4], xnsr[:, H4:2 * H4]
                              + b12_ref[...]], axis=0)
        db = jnp.concatenate([xnsr[:, 2 * H4:3 * H4], xnsr[:, 3 * H4:]
                              + b12_ref[...]], axis=0)
        xnstack = jnp.concatenate([da, db], axis=1).astype(bf16)
        pre2 = (jnp.dot(src_ref[...], xnstack, preferred_element_type=f32)
                + jnp.dot(msg, w1k2_ref[...], preferred_element_type=f32))
        h2 = _elu(pre2.astype(bf16))

        y2 = jnp.dot(h2, w22_ref[...], preferred_element_type=f32)
        res = (_elu(y2.astype(bf16) + b22_ref[...]) * sc2_ref[...]
               + sh2_ref[...])
        Dout = res.shape[1] // 2
        o_ref[0] = res[:, :Dout].astype(f32)
        res_ref[...] = res[:, Dout:]

    @pl.when(t == 1)
    def _flush():
        o_ref[0] = res_ref[...].astype(jnp.float32)


def kernel(emb_w1, emb_b1, emb_w2, emb_b2, emb_gamma, emb_beta,
           e1_w1, e1_b1, e1_w2, e1_b2, e1_gamma, e1_beta,
           n1_w1, n1_b1, n1_w2, n1_b2, n1_gamma, n1_beta,
           e2_w1, e2_b1, e2_w2, e2_b2, e2_gamma, e2_beta,
           inputs, rel_rec, rel_send):
    f32 = jnp.float32
    bf16 = jnp.bfloat16
    B, N, n_in = inputs.shape
    E = rel_rec.shape[0]
    D = emb_w2.shape[1]
    Dn = n1_w2.shape[1]
    Dout = e2_w2.shape[1]

    sq = jnp.sqrt(jnp.asarray(1.0 + BN_EPS, f32))
    sce, she = emb_gamma / sq, emb_beta
    sc1, sh1 = e1_gamma / sq, e1_beta
    scn, shn = n1_gamma / sq, n1_beta
    sc2, sh2 = e2_gamma / sq, e2_beta

    src_cat = jnp.concatenate([rel_send, rel_rec], axis=1).astype(bf16)
    rt = rel_rec.T.astype(bf16)

    w1sr1 = jnp.concatenate([e1_w1[:D], e1_w1[D:]], axis=1)
    w1sr1_eff = sce[:, None] * w1sr1
    b11_eff = e1_b1 + (she @ w1sr1)[:D] + (she @ w1sr1)[D:]
    wn1_eff = (sc1[:, None] * n1_w1) / float(N)
    bn1_eff = n1_b1 + (N - 1) / float(N) * (sh1 @ n1_w1)
    w1sr2 = jnp.concatenate([e2_w1[:Dn], e2_w1[Dn:2 * Dn]], axis=1)
    w1sr2_eff = scn[:, None] * w1sr2
    w1k_eff = sc1[:, None] * e2_w1[2 * Dn:]
    b12_eff = (e2_b1 + sh1 @ e2_w1[2 * Dn:]
               + (shn @ w1sr2)[:Dn] + (shn @ w1sr2)[Dn:])

    def bdiag(w):
        z = jnp.zeros_like(w)
        return jnp.block([[w, z], [z, w]])

    pair = lambda v: jnp.tile(v.reshape(1, -1), (1, 2))

    args = (
        inputs, src_cat, rt,
        emb_w1.astype(bf16), emb_b1.reshape(1, -1),
        emb_w2.astype(bf16), emb_b2.reshape(1, -1),
        w1sr1_eff.astype(bf16), b11_eff.reshape(1, -1),
        bdiag(e1_w2).astype(bf16), pair(e1_b2).astype(bf16),
        bdiag(wn1_eff).astype(bf16), pair(bn1_eff),
        bdiag(n1_w2).astype(bf16), pair(n1_b2),
        bdiag(w1sr2_eff).astype(bf16), bdiag(w1k_eff).astype(bf16),
        b12_eff.reshape(1, -1),
        bdiag(e2_w2).astype(bf16), pair(e2_b2).astype(bf16),
        pair(sc2).astype(bf16), pair(sh2).astype(bf16),
    )

    const2 = lambda p, t: (0, 0)
    in_specs = [pl.BlockSpec((2, N, n_in), lambda p, t: (p, 0, 0))]
    in_specs += [pl.BlockSpec(a.shape, const2) for a in args[1:]]

    return pl.pallas_call(
        _fused_kernel,
        out_shape=jax.ShapeDtypeStruct((B, E, Dout), f32),
        grid=(B // 2, 2),
        in_specs=in_specs,
        out_specs=pl.BlockSpec((1, E, Dout), lambda p, t: (2 * p + t, 0, 0)),
        scratch_shapes=[pltpu.VMEM((E, Dout), bf16)],
        compiler_params=pltpu.CompilerParams(
            dimension_semantics=("arbitrary", "arbitrary"),
            vmem_limit_bytes=VMEM_LIMIT),
    )(*args)


# speedup vs baseline: 2.2489x; 1.0064x over previous
import jax
import jax.numpy as jnp
from jax.experimental import pallas as pl
from jax.experimental.pallas import tpu as pltpu

BN_EPS = 1e-5
VMEM_LIMIT = 110 * 1024 * 1024


def _elu(x):
    one = jnp.asarray(1.0, x.dtype)
    return jnp.maximum(x, jnp.exp(jnp.minimum(x, 0)) - one)


def _fused_kernel(x_ref, src_ref, rt_ref,
                  we1_ref, be1_ref, we2_ref, be2_ref,
                  w1sr1_ref, b11_ref, w21_ref, b21_ref,
                  wn1_ref, bn1_ref, wn2_ref, bn2_ref,
                  w1sr2_ref, w1k2_ref, b12_ref, w22_ref, b22_ref,
                  sc2_ref, sh2_ref,
                  o_ref, res_ref, sem_a, sem_b):
    f32 = jnp.float32
    bf16 = jnp.bfloat16
    N = x_ref.shape[1]
    p = pl.program_id(0)
    np_ = pl.num_programs(0)

    xin = x_ref[...].reshape(2 * N, x_ref.shape[2]).astype(bf16)
    h = _elu(jnp.dot(xin, we1_ref[...], preferred_element_type=f32)
             + be1_ref[...])
    y = _elu(jnp.dot(h.astype(bf16), we2_ref[...],
                     preferred_element_type=f32) + be2_ref[...])
    x = y.astype(bf16)

    xsr = jnp.dot(x, w1sr1_ref[...], preferred_element_type=f32)
    H = xsr.shape[1] // 2
    ca = jnp.concatenate([xsr[:N, :H], xsr[:N, H:] + b11_ref[...]],
                         axis=0)
    cb = jnp.concatenate([xsr[N:, :H], xsr[N:, H:] + b11_ref[...]],
                         axis=0)
    xstack = jnp.concatenate([ca, cb], axis=1).astype(bf16)
    pre1 = jnp.dot(src_ref[...], xstack, preferred_element_type=f32)
    h1 = _elu(pre1.astype(bf16))

    m1 = jnp.dot(h1, w21_ref[...], preferred_element_type=f32)
    msg = _elu(m1.astype(bf16) + b21_ref[...])

    aggraw = jnp.dot(rt_ref[...], msg, preferred_element_type=f32)

    hn = _elu(jnp.dot(aggraw.astype(bf16), wn1_ref[...],
                      preferred_element_type=f32) + bn1_ref[...])
    yn = _elu(jnp.dot(hn.astype(bf16), wn2_ref[...],
                      preferred_element_type=f32) + bn2_ref[...])
    xn = yn.astype(bf16)

    xnsr = jnp.dot(xn, w1sr2_ref[...], preferred_element_type=f32)
    H4 = xnsr.shape[1] // 4
    da = jnp.concatenate([xnsr[:, :H4], xnsr[:, H4:2 * H4]
                          + b12_ref[...]], axis=0)
    db = jnp.concatenate([xnsr[:, 2 * H4:3 * H4], xnsr[:, 3 * H4:]
                          + b12_ref[...]], axis=0)
    xnstack = jnp.concatenate([da, db], axis=1).astype(bf16)
    pre2 = (jnp.dot(src_ref[...], xnstack, preferred_element_type=f32)
            + jnp.dot(msg, w1k2_ref[...], preferred_element_type=f32))
    h2 = _elu(pre2.astype(bf16))

    y2 = (jnp.dot(h2, w22_ref[...], preferred_element_type=f32)
          + b22_ref[...])
    res = _elu(y2) * sc2_ref[...] + sh2_ref[...]
    Dout = res.shape[1] // 2

    @pl.when(p > 0)
    def _wait_prev():
        pltpu.make_async_copy(res_ref.at[0], o_ref.at[0], sem_a).wait()
        pltpu.make_async_copy(res_ref.at[1], o_ref.at[1], sem_b).wait()

    res_ref[0] = res[:, :Dout]
    res_ref[1] = res[:, Dout:]
    cp_a = pltpu.make_async_copy(res_ref.at[0], o_ref.at[2 * p], sem_a)
    cp_b = pltpu.make_async_copy(res_ref.at[1], o_ref.at[2 * p + 1], sem_b)
    cp_a.start()
    cp_b.start()

    @pl.when(p == np_ - 1)
    def _wait_last():
        cp_a.wait()
        cp_b.wait()


def kernel(emb_w1, emb_b1, emb_w2, emb_b2, emb_gamma, emb_beta,
           e1_w1, e1_b1, e1_w2, e1_b2, e1_gamma, e1_beta,
           n1_w1, n1_b1, n1_w2, n1_b2, n1_gamma, n1_beta,
           e2_w1, e2_b1, e2_w2, e2_b2, e2_gamma, e2_beta,
           inputs, rel_rec, rel_send):
    f32 = jnp.float32
    bf16 = jnp.bfloat16
    B, N, n_in = inputs.shape
    E = rel_rec.shape[0]
    D = emb_w2.shape[1]
    Dn = n1_w2.shape[1]
    Dout = e2_w2.shape[1]

    sq = jnp.sqrt(jnp.asarray(1.0 + BN_EPS, f32))
    sce, she = emb_gamma / sq, emb_beta
    sc1, sh1 = e1_gamma / sq, e1_beta
    scn, shn = n1_gamma / sq, n1_beta
    sc2, sh2 = e2_gamma / sq, e2_beta

    src_cat = jnp.concatenate([rel_send, rel_rec], axis=1).astype(bf16)
    rt = rel_rec.T.astype(bf16)

    w1sr1 = jnp.concatenate([e1_w1[:D], e1_w1[D:]], axis=1)
    w1sr1_eff = sce[:, None] * w1sr1
    b11_eff = e1_b1 + (she @ w1sr1)[:D] + (she @ w1sr1)[D:]
    wn1_eff = (sc1[:, None] * n1_w1) / float(N)
    bn1_eff = n1_b1 + (N - 1) / float(N) * (sh1 @ n1_w1)
    w1sr2 = jnp.concatenate([e2_w1[:Dn], e2_w1[Dn:2 * Dn]], axis=1)
    w1sr2_eff = scn[:, None] * w1sr2
    w1k_eff = sc1[:, None] * e2_w1[2 * Dn:]
    b12_eff = (e2_b1 + sh1 @ e2_w1[2 * Dn:]
               + (shn @ w1sr2)[:Dn] + (shn @ w1sr2)[Dn:])

    def bdiag(w):
        z = jnp.zeros_like(w)
        return jnp.block([[w, z], [z, w]])

    pair = lambda v: jnp.tile(v.reshape(1, -1), (1, 2))

    args = (
        inputs, src_cat, rt,
        emb_w1.astype(bf16), emb_b1.reshape(1, -1),
        emb_w2.astype(bf16), emb_b2.reshape(1, -1),
        w1sr1_eff.astype(bf16), b11_eff.reshape(1, -1),
        bdiag(e1_w2).astype(bf16), pair(e1_b2).astype(bf16),
        bdiag(wn1_eff).astype(bf16), pair(bn1_eff),
        bdiag(n1_w2).astype(bf16), pair(n1_b2),
        bdiag(w1sr2_eff).astype(bf16), bdiag(w1k_eff).astype(bf16),
        b12_eff.reshape(1, -1),
        bdiag(e2_w2).astype(bf16), pair(e2_b2),
        pair(sc2), pair(sh2),
    )

    const2 = lambda p: (0, 0)
    in_specs = [pl.BlockSpec((2, N, n_in), lambda p: (p, 0, 0))]
    in_specs += [pl.BlockSpec(a.shape, const2) for a in args[1:]]

    return pl.pallas_call(
        _fused_kernel,
        out_shape=jax.ShapeDtypeStruct((B, E, Dout), f32),
        grid=(B // 2,),
        in_specs=in_specs,
        out_specs=pl.BlockSpec(memory_space=pl.ANY),
        scratch_shapes=[pltpu.VMEM((2, E, Dout), f32),
                        pltpu.SemaphoreType.DMA,
                        pltpu.SemaphoreType.DMA],
        compiler_params=pltpu.CompilerParams(
            dimension_semantics=("arbitrary",),
            vmem_limit_bytes=VMEM_LIMIT),
    )(*args)
